# Initial kernel scaffold; baseline (speedup 1.0000x reference)
import jax
import jax.numpy as jnp
from jax.experimental import pallas as pl


def kernel(queries, positions, scales, quaternions):
    raise NotImplementedError("write your pallas kernel here")


# speedup vs baseline: 3.9702x; 3.9702x over previous
import jax
import jax.numpy as jnp
from jax import lax
from jax.experimental import pallas as pl

K = 8
BQ = 128


def _quat_rotmat(q):
    q = q / (jnp.linalg.norm(q, axis=-1, keepdims=True) + 1e-12)
    w, x, y, z = q[..., 0], q[..., 1], q[..., 2], q[..., 3]
    R = jnp.stack([
        1 - 2 * (y * y + z * z), 2 * (x * y - w * z), 2 * (x * z + w * y),
        2 * (x * y + w * z), 1 - 2 * (x * x + z * z), 2 * (y * z - w * x),
        2 * (x * z - w * y), 2 * (y * z + w * x), 1 - 2 * (x * x + y * y)
    ], axis=-1).reshape(q.shape[:-1] + (3, 3))
    return R


def _topk_body(qfeat_ref, q_ref, sflat_ref, b_ref, c_ref, vals_ref, idx_ref):
    n = sflat_ref.shape[0]
    term1 = lax.dot_general(qfeat_ref[...].astype(jnp.bfloat16),
                            sflat_ref[...],
                            (((1,), (1,)), ((), ())),
                            preferred_element_type=jnp.float32)
    term2 = -2.0 * lax.dot_general(q_ref[...].astype(jnp.bfloat16),
                                   b_ref[...].astype(jnp.bfloat16),
                                   (((1,), (1,)), ((), ())),
                                   preferred_element_type=jnp.float32)
    dist = (term1 + term2) + c_ref[...]
    iota = lax.broadcasted_iota(jnp.int32, (BQ, n), 1)
    for k in range(K):
        m = jnp.min(dist, axis=1, keepdims=True)
        am = jnp.min(jnp.where(dist == m, iota, n), axis=1, keepdims=True)
        vals_ref[:, k:k + 1] = m
        idx_ref[:, k:k + 1] = am
        if k < K - 1:
            dist = jnp.where(iota == am, jnp.inf, dist)


def kernel(queries, positions, scales, quaternions):
    Q = queries.shape[0]
    N = positions.shape[0]
    Rm = _quat_rotmat(quaternions)
    s2inv = 1.0 / (scales * scales + 1e-8)
    A = Rm * s2inv[:, None, :]
    Abf = A.astype(jnp.bfloat16)
    Rbf = Rm.astype(jnp.bfloat16)
    Sinv = jnp.einsum('nij,nkj->nik', Rbf, Abf,
                      preferred_element_type=jnp.float32)
    Sflat_bf = Sinv.astype(jnp.bfloat16).reshape(N, 9)
    bt = [Sinv[:, :, j] * positions[:, None, j] for j in range(3)]
    b = (bt[0] + bt[1]) + bt[2]
    ct = [positions[:, j] * b[:, j] for j in range(3)]
    c = (ct[0] + ct[2]) + ct[1]
    Qfeat = (queries[:, :, None] * queries[:, None, :]).reshape(Q, 9)

    grid = (Q // BQ,)
    vals, idx = pl.pallas_call(
        _topk_body,
        grid=grid,
        in_specs=[
            pl.BlockSpec((BQ, 9), lambda i: (i, 0)),
            pl.BlockSpec((BQ, 3), lambda i: (i, 0)),
            pl.BlockSpec((N, 9), lambda i: (0, 0)),
            pl.BlockSpec((N, 3), lambda i: (0, 0)),
            pl.BlockSpec((1, N), lambda i: (0, 0)),
        ],
        out_specs=[
            pl.BlockSpec((BQ, K), lambda i: (i, 0)),
            pl.BlockSpec((BQ, K), lambda i: (i, 0)),
        ],
        out_shape=[
            jax.ShapeDtypeStruct((Q, K), jnp.float32),
            jax.ShapeDtypeStruct((Q, K), jnp.int32),
        ],
    )(Qfeat, queries, Sflat_bf, b, c[None, :])
    return vals, idx
